# Initial kernel scaffold; baseline (speedup 1.0000x reference)
import jax
import jax.numpy as jnp
from jax.experimental import pallas as pl


def kernel(z, codebook_weight):
    raise NotImplementedError("write your pallas kernel here")


# speedup vs baseline: 4.4138x; 4.4138x over previous
import functools

import jax
import jax.numpy as jnp
from jax import lax
from jax.experimental import pallas as pl
from jax.experimental.pallas import tpu as pltpu
from jax.experimental.pallas import tpu_sc as plsc

N_CB = 8192
D = 32
N_TOK = 16384
TT = 512
CC = 1024
N_TILES = N_TOK // TT
SCALE = 1.0 / 16384.0
NC = 2
NS = 16
NW = NC * NS
ROWS_PER_W = N_TOK // NW
G_CHUNK = 128
H_ROWS = N_TOK // NS // G_CHUNK


def _tc_body(z_ref, zsq_ref, e_ref, esq_ref, idx_ref, idxr_ref, loss_ref, d_scr):
    t = pl.program_id(0)
    zt = z_ref[...]
    zb = zt.astype(jnp.bfloat16).astype(jnp.float32)
    a = zsq_ref[...]
    BIG = jnp.int32(2 ** 30)
    rw = rq = rc = rqr = rcr = rmin = None
    for k in range(N_CB // CC):
        ec = e_ref[k * CC:(k + 1) * CC, :]
        b = esq_ref[:, k * CC:(k + 1) * CC]
        s = lax.dot_general(zb, ec, (((1,), (1,)), ((), ())),
                            preferred_element_type=jnp.float32)
        d_scr[...] = (a + b) - 2.0 * s
        d = d_scr[...]
        w = d.astype(jnp.bfloat16).astype(jnp.float32)
        wm = jnp.min(w, axis=1, keepdims=True)
        ii = lax.broadcasted_iota(jnp.int32, (TT, CC), 1) + (k * CC)
        att = w == wm
        blw = d < wm
        qk = jnp.min(jnp.where(att, ii, BIG), axis=1, keepdims=True)
        qkr = jnp.max(jnp.where(att, ii, jnp.int32(-1)), axis=1, keepdims=True)
        ck = jnp.max(jnp.where(blw, ii, jnp.int32(-1)), axis=1, keepdims=True)
        ckr = jnp.min(jnp.where(blw, ii, BIG), axis=1, keepdims=True)
        mk = jnp.min(d, axis=1, keepdims=True)
        if k == 0:
            rw, rq, rc, rqr, rcr, rmin = wm, qk, ck, qkr, ckr, mk
        else:
            better = wm < rw
            eqw = wm == rw
            rq = jnp.where(better, qk, rq)
            rc = jnp.where(better, ck, jnp.where(eqw, jnp.maximum(rc, ck), rc))
            rqr = jnp.where(better, qkr, jnp.where(eqw, jnp.maximum(rqr, qkr), rqr))
            rcr = jnp.where(better, ckr, jnp.where(eqw, jnp.minimum(rcr, ckr), rcr))
            rw = jnp.where(better, wm, rw)
            rmin = jnp.minimum(rmin, mk)
    idx_ref[0] = jnp.maximum(rq, rc)
    idxr_ref[0] = jnp.minimum(rqr, rcr)

    @pl.when(t == 0)
    def _init():
        loss_ref[...] = jnp.zeros((1, 1), jnp.float32)

    loss_ref[...] += jnp.sum(rmin, axis=0, keepdims=True)

    @pl.when(t == N_TILES - 1)
    def _fini():
        loss_ref[...] *= jnp.float32(1.0 / (N_TOK * D))


def _sc_body(e_hbm, idx_hbm, zq_hbm, hist_hbm,
             idxg, rows, idxh, ones, hv, zv, shared, sem):
    cid = lax.axis_index("c")
    sid = lax.axis_index("s")
    wid = sid * NC + cid

    pltpu.sync_copy(idx_hbm.at[pl.ds(wid * 4, 4)], idxg)
    cps = [pltpu.async_copy(e_hbm.at[idxg.at[j]],
                            rows.at[pl.ds(j * G_CHUNK, G_CHUNK)], sem)
           for j in range(4)]
    for cp in cps:
        cp.wait()
    pltpu.sync_copy(rows, zq_hbm.at[pl.ds(wid * ROWS_PER_W, ROWS_PER_W)])

    @pl.when(cid == 0)
    def _hist():
        pltpu.sync_copy(idx_hbm.at[pl.ds(sid * H_ROWS, H_ROWS)], idxh)
        for i in range(G_CHUNK // 16):
            ones[pl.ds(i * 16, 16)] = jnp.full((16,), 1.0, jnp.float32)
        for i in range(512 // 16):
            zv[pl.ds(i * 16, 16)] = jnp.zeros((16,), jnp.float32)
        pltpu.sync_copy(zv, shared.at[pl.ds(sid * 512, 512)])
        plsc.subcore_barrier()
        for j in range(H_ROWS):
            pltpu.sync_copy(ones, shared.at[idxh.at[j]], add=True)
        plsc.subcore_barrier()
        pltpu.sync_copy(shared.at[pl.ds(sid * 512, 512)], hv)
        for i in range(512 // 16):
            hv[pl.ds(i * 16, 16)] = hv[pl.ds(i * 16, 16)] * jnp.float32(SCALE)
        pltpu.sync_copy(hv, hist_hbm.at[pl.ds(sid * 512, 512)])


@functools.lru_cache(maxsize=1)
def _sc_call():
    mesh = plsc.VectorSubcoreMesh(core_axis_name="c", subcore_axis_name="s",
                                  num_cores=NC)
    return pl.kernel(
        _sc_body,
        mesh=mesh,
        compiler_params=pltpu.CompilerParams(use_tc_tiling_on_sc=False),
        out_type=[
            jax.ShapeDtypeStruct((N_TOK, D), jnp.float32),
            jax.ShapeDtypeStruct((N_CB,), jnp.float32),
        ],
        scratch_types=[
            pltpu.VMEM((4, G_CHUNK), jnp.int32),
            pltpu.VMEM((ROWS_PER_W, D), jnp.float32),
            pltpu.VMEM((H_ROWS, G_CHUNK), jnp.int32),
            pltpu.VMEM((G_CHUNK,), jnp.float32),
            pltpu.VMEM((512,), jnp.float32),
            pltpu.VMEM((512,), jnp.float32),
            pltpu.VMEM_SHARED((N_CB,), jnp.float32),
            pltpu.SemaphoreType.DMA,
        ],
    )


def kernel(z, codebook_weight):
    e = lax.stop_gradient(codebook_weight)
    z_flat = z.reshape(-1, e.shape[1])
    zsq = jnp.sum(z_flat ** 2, axis=1, keepdims=True)
    esq = jnp.sum(e ** 2, axis=1)

    idx3, idx3r, loss = pl.pallas_call(
        _tc_body,
        grid=(N_TILES,),
        in_specs=[
            pl.BlockSpec((TT, D), lambda t: (t, 0)),
            pl.BlockSpec((TT, 1), lambda t: (t, 0)),
            pl.BlockSpec((N_CB, D), lambda t: (0, 0)),
            pl.BlockSpec((1, N_CB), lambda t: (0, 0)),
        ],
        out_specs=[
            pl.BlockSpec((1, TT, 1), lambda t: (t, 0, 0)),
            pl.BlockSpec((1, TT, 1), lambda t: (t, 0, 0)),
            pl.BlockSpec((1, 1), lambda t: (0, 0)),
        ],
        out_shape=[
            jax.ShapeDtypeStruct((N_TILES, TT, 1), jnp.int32),
            jax.ShapeDtypeStruct((N_TILES, TT, 1), jnp.int32),
            jax.ShapeDtypeStruct((1, 1), jnp.float32),
        ],
        scratch_shapes=[pltpu.VMEM((TT, CC), jnp.float32)],
    )(z_flat, zsq, e, esq.reshape(1, N_CB))

    zero = jnp.min(z_flat) * 0.0
    x_hi = jnp.float32(1.0 - 2.0 ** -11)
    x_lo = jnp.float32(1.0 - 2.0 ** -11 - 2.0 ** -13)
    col = jnp.arange(N_CB, dtype=jnp.int32)[None, :]
    parr = jnp.where(col == 0, x_hi,
                     jnp.where(col == N_CB - 1, x_lo, jnp.float32(2.0)))
    parr = jnp.broadcast_to(parr, (16, N_CB)) + zero
    probe_idx = jnp.argmin(parr, axis=1)[0]
    is_fwd = probe_idx == jnp.int32(N_CB - 1)

    min_idx = jnp.where(is_fwd, idx3.reshape(N_TOK), idx3r.reshape(N_TOK))
    zq_flat, hist = _sc_call()(e, min_idx.reshape(N_TOK // G_CHUNK, G_CHUNK))

    commitment_loss = loss.reshape(())
    z_q_st = zq_flat.reshape(z.shape)
    encoding_indices = min_idx.reshape(z.shape[0], z.shape[1])
    return (commitment_loss, z_q_st, encoding_indices, hist)
